# Initial kernel scaffold; baseline (speedup 1.0000x reference)
import jax
import jax.numpy as jnp
from jax.experimental import pallas as pl
from jax.experimental.pallas import tpu as pltpu

_N = 16384
_E = 32768
_D = 128

_TE_EDGE = 4096
_TN = 512
_TK = 2048


def _edge_kernel(s0_ref, d0_ref, e0_ref, s1_ref, d1_ref, e1_ref,
                 w1_ref, b1_ref, w2_ref, b2_ref, g_ref, bt_ref,
                 p0_ref, p1_ref, r0_ref, r1_ref, xcat_ref):
    d = _D
    e0f = e0_ref[...]
    e1f = e1_ref[...]
    xcat_ref[:, 0:d] = s0_ref[...]
    xcat_ref[:, d:2 * d] = d0_ref[...]
    xcat_ref[:, 2 * d:3 * d] = e0f.astype(jnp.bfloat16)
    xcat_ref[:, 3 * d:4 * d] = s1_ref[...]
    xcat_ref[:, 4 * d:5 * d] = d1_ref[...]
    xcat_ref[:, 5 * d:6 * d] = e1f.astype(jnp.bfloat16)

    h = jnp.dot(xcat_ref[...], w1_ref[...],
                preferred_element_type=jnp.float32) + b1_ref[...]
    h = jnp.maximum(h, 0.0)
    y = jnp.dot(h.astype(jnp.bfloat16), w2_ref[...],
                preferred_element_type=jnp.float32) + b2_ref[...]

    for s, (p_ref, r_ref, ef) in enumerate(((p0_ref, r0_ref, e0f),
                                            (p1_ref, r1_ref, e1f))):
        ys = y[:, s * d:(s + 1) * d]
        mu = jnp.mean(ys, axis=-1, keepdims=True)
        var = jnp.maximum(jnp.mean(ys * ys, axis=-1, keepdims=True) - mu * mu,
                          0.0)
        ln = (ys - mu) * jax.lax.rsqrt(var + 1e-5) \
            * g_ref[:, s * d:(s + 1) * d] + bt_ref[:, s * d:(s + 1) * d]
        p_ref[...] = ln.astype(jnp.bfloat16)
        r_ref[...] = ln + ef


def _edge_mlps(s0, d0, e0, s1, d1, e1, w1bd, b1c, w2bd, b2c, gc, btc):
    te = _TE_EDGE
    row = lambda ei: (ei, 0)
    return pl.pallas_call(
        _edge_kernel,
        out_shape=(jax.ShapeDtypeStruct((_E, _D), jnp.bfloat16),
                   jax.ShapeDtypeStruct((_E, _D), jnp.bfloat16),
                   jax.ShapeDtypeStruct((_E, _D), jnp.float32),
                   jax.ShapeDtypeStruct((_E, _D), jnp.float32)),
        grid=(_E // te,),
        in_specs=[pl.BlockSpec((te, _D), row)] * 6 + [
            pl.BlockSpec((6 * _D, 2 * _D), lambda ei: (0, 0)),
            pl.BlockSpec((1, 2 * _D), lambda ei: (0, 0)),
            pl.BlockSpec((2 * _D, 2 * _D), lambda ei: (0, 0)),
            pl.BlockSpec((1, 2 * _D), lambda ei: (0, 0)),
            pl.BlockSpec((1, 2 * _D), lambda ei: (0, 0)),
            pl.BlockSpec((1, 2 * _D), lambda ei: (0, 0)),
        ],
        out_specs=[pl.BlockSpec((te, _D), row)] * 4,
        scratch_shapes=[pltpu.VMEM((te, 6 * _D), jnp.bfloat16)],
        compiler_params=pltpu.CompilerParams(
            dimension_semantics=("parallel",)),
    )(s0, d0, e0, s1, d1, e1, w1bd, b1c, w2bd, b2c, gc, btc)


def _node_kernel(ndt_ref, rc0_ref, rc1_ref, em0_ref, em1_ref,
                 w1_ref, b1t_ref, w2_ref, b2t_ref, gt_ref, btt_ref,
                 ot_ref, xcat_ref):
    n_base = pl.program_id(0) * _TN
    row_ids = jax.lax.broadcasted_iota(jnp.int32, (_TN, _TK), 0) + n_base

    one = jnp.bfloat16(1.0)
    zero = jnp.bfloat16(0.0)
    dn = (((0,), (1,)), ((), ()))

    for s, (rc_ref, em_ref) in enumerate(((rc0_ref, em0_ref),
                                          (rc1_ref, em1_ref))):
        acc = jnp.zeros((_D, _TN), jnp.float32)
        for kb in range(_E // _TK):
            eo = kb * _TK
            rcv = rc_ref[0:1, eo:eo + _TK]
            onehot = jnp.where(row_ids == rcv, one, zero)
            acc = acc + jax.lax.dot_general(
                em_ref[eo:eo + _TK, :], onehot, dn,
                preferred_element_type=jnp.float32)
        xcat_ref[(s + 1) * _D:(s + 2) * _D, :] = acc.astype(jnp.bfloat16)

    xt = ndt_ref[...]
    xcat_ref[0:_D, :] = xt.astype(jnp.bfloat16)

    dt = (((0,), (0,)), ((), ()))
    h = jax.lax.dot_general(w1_ref[...], xcat_ref[...], dt,
                            preferred_element_type=jnp.float32) + b1t_ref[...]
    h = jnp.maximum(h, 0.0)
    y = jax.lax.dot_general(w2_ref[...], h.astype(jnp.bfloat16), dt,
                            preferred_element_type=jnp.float32) + b2t_ref[...]
    mu = jnp.mean(y, axis=0, keepdims=True)
    var = jnp.maximum(jnp.mean(y * y, axis=0, keepdims=True) - mu * mu, 0.0)
    yn = (y - mu) * jax.lax.rsqrt(var + 1e-5)
    ot_ref[...] = yn * gt_ref[...] + btt_ref[...] + xt


def _node_update(ndt, rc0, rc1, em0, em1, w1, b1t, w2, b2t, gt, btt):
    def full2(shp):
        return pl.BlockSpec(shp, lambda i: (0, 0))
    return pl.pallas_call(
        _node_kernel,
        out_shape=jax.ShapeDtypeStruct((_D, _N), jnp.float32),
        grid=(_N // _TN,),
        in_specs=[
            pl.BlockSpec((_D, _TN), lambda i: (0, i)),
            full2((1, _E)), full2((1, _E)),
            full2((_E, _D)), full2((_E, _D)),
            full2((3 * _D, _D)),
            full2((_D, _TN)),
            full2((_D, _D)),
            full2((_D, _TN)),
            full2((_D, _TN)),
            full2((_D, _TN)),
        ],
        out_specs=pl.BlockSpec((_D, _TN), lambda i: (0, i)),
        scratch_shapes=[pltpu.VMEM((3 * _D, _TN), jnp.bfloat16)],
        compiler_params=pltpu.CompilerParams(
            dimension_semantics=("parallel",)),
    )(ndt, rc0, rc1, em0, em1, w1, b1t, w2, b2t, gt, btt)


def kernel(node_features, ef0, snd0, rcv0, ef1, snd1, rcv1,
           edge_w1, edge_b1, edge_w2, edge_b2, edge_gamma, edge_beta,
           node_w1, node_b1, node_w2, node_b2, node_gamma, node_beta):
    d = _D
    node_bf = node_features.astype(jnp.bfloat16)
    s0 = jnp.take(node_bf, snd0.astype(jnp.int32), axis=0)
    d0 = jnp.take(node_bf, rcv0.astype(jnp.int32), axis=0)
    s1 = jnp.take(node_bf, snd1.astype(jnp.int32), axis=0)
    d1 = jnp.take(node_bf, rcv1.astype(jnp.int32), axis=0)

    zw1 = jnp.zeros((3 * d, d), jnp.bfloat16)
    w1bd = jnp.concatenate([
        jnp.concatenate([edge_w1[0], zw1], axis=1),
        jnp.concatenate([zw1, edge_w1[1]], axis=1)], axis=0)
    zw2 = jnp.zeros((d, d), jnp.bfloat16)
    w2bd = jnp.concatenate([
        jnp.concatenate([edge_w2[0], zw2], axis=1),
        jnp.concatenate([zw2, edge_w2[1]], axis=1)], axis=0)
    b1c = jnp.concatenate([edge_b1[0], edge_b1[1]], axis=1)
    b2c = jnp.concatenate([edge_b2[0], edge_b2[1]], axis=1)
    gc = jnp.concatenate([edge_gamma[0], edge_gamma[1]], axis=1)
    btc = jnp.concatenate([edge_beta[0], edge_beta[1]], axis=1)

    em0, em1, ne0, ne1 = _edge_mlps(s0, d0, ef0, s1, d1, ef1,
                                    w1bd, b1c, w2bd, b2c, gc, btc)

    ndt = node_features.T
    rc0 = rcv0.astype(jnp.int32).reshape(1, _E)
    rc1 = rcv1.astype(jnp.int32).reshape(1, _E)
    b1t = jnp.broadcast_to(node_b1.reshape(d, 1), (d, _TN))
    b2t = jnp.broadcast_to(node_b2.reshape(d, 1), (d, _TN))
    gt = jnp.broadcast_to(node_gamma.reshape(d, 1), (d, _TN))
    btt = jnp.broadcast_to(node_beta.reshape(d, 1), (d, _TN))

    out_t = _node_update(ndt, rc0, rc1, em0, em1,
                         node_w1, b1t, node_w2, b2t, gt, btt)
    return out_t.T, [ne0, ne1]


# speedup vs baseline: 1.2639x; 1.2639x over previous
import jax
import jax.numpy as jnp
from jax.experimental import pallas as pl
from jax.experimental.pallas import tpu as pltpu

_N = 16384
_E = 32768
_D = 128

_TE_EDGE = 4096
_TN = 512
_TK = 2048


def _edge_kernel(s0_ref, d0_ref, e0_ref, s1_ref, d1_ref, e1_ref,
                 w1_ref, b1_ref, w2_ref, b2_ref, g_ref, bt_ref,
                 p0_ref, p1_ref, r0_ref, r1_ref, xcat_ref):
    d = _D
    e0f = e0_ref[...]
    e1f = e1_ref[...]
    xcat_ref[:, 0:d] = s0_ref[...]
    xcat_ref[:, d:2 * d] = d0_ref[...]
    xcat_ref[:, 2 * d:3 * d] = e0f.astype(jnp.bfloat16)
    xcat_ref[:, 3 * d:4 * d] = s1_ref[...]
    xcat_ref[:, 4 * d:5 * d] = d1_ref[...]
    xcat_ref[:, 5 * d:6 * d] = e1f.astype(jnp.bfloat16)

    h = jnp.dot(xcat_ref[...], w1_ref[...],
                preferred_element_type=jnp.float32) + b1_ref[...]
    h = jnp.maximum(h, 0.0)
    y = jnp.dot(h.astype(jnp.bfloat16), w2_ref[...],
                preferred_element_type=jnp.float32) + b2_ref[...]

    for s, (p_ref, r_ref, ef) in enumerate(((p0_ref, r0_ref, e0f),
                                            (p1_ref, r1_ref, e1f))):
        ys = y[:, s * d:(s + 1) * d]
        mu = jnp.mean(ys, axis=-1, keepdims=True)
        var = jnp.maximum(jnp.mean(ys * ys, axis=-1, keepdims=True) - mu * mu,
                          0.0)
        ln = (ys - mu) * jax.lax.rsqrt(var + 1e-5) \
            * g_ref[:, s * d:(s + 1) * d] + bt_ref[:, s * d:(s + 1) * d]
        p_ref[...] = ln.astype(jnp.bfloat16)
        r_ref[...] = ln + ef


def _edge_mlps(s0, d0, e0, s1, d1, e1, w1bd, b1c, w2bd, b2c, gc, btc):
    te = _TE_EDGE
    row = lambda ei: (ei, 0)
    return pl.pallas_call(
        _edge_kernel,
        out_shape=(jax.ShapeDtypeStruct((_E, _D), jnp.bfloat16),
                   jax.ShapeDtypeStruct((_E, _D), jnp.bfloat16),
                   jax.ShapeDtypeStruct((_E, _D), jnp.float32),
                   jax.ShapeDtypeStruct((_E, _D), jnp.float32)),
        grid=(_E // te,),
        in_specs=[pl.BlockSpec((te, _D), row)] * 6 + [
            pl.BlockSpec((6 * _D, 2 * _D), lambda ei: (0, 0)),
            pl.BlockSpec((1, 2 * _D), lambda ei: (0, 0)),
            pl.BlockSpec((2 * _D, 2 * _D), lambda ei: (0, 0)),
            pl.BlockSpec((1, 2 * _D), lambda ei: (0, 0)),
            pl.BlockSpec((1, 2 * _D), lambda ei: (0, 0)),
            pl.BlockSpec((1, 2 * _D), lambda ei: (0, 0)),
        ],
        out_specs=[pl.BlockSpec((te, _D), row)] * 4,
        scratch_shapes=[pltpu.VMEM((te, 6 * _D), jnp.bfloat16)],
        compiler_params=pltpu.CompilerParams(
            dimension_semantics=("parallel",)),
    )(s0, d0, e0, s1, d1, e1, w1bd, b1c, w2bd, b2c, gc, btc)


def _node_kernel(ndt_ref, rc0_ref, rc1_ref, em0_ref, em1_ref,
                 w1_ref, b1t_ref, w2_ref, b2t_ref, gt_ref, btt_ref,
                 ot_ref, xcat_ref):
    n_base = (pl.program_id(0) * _TN).astype(jnp.int16)
    row_ids = jax.lax.broadcasted_iota(jnp.int16, (_TN, _TK), 0) + n_base

    one = jnp.bfloat16(1.0)
    zero = jnp.bfloat16(0.0)
    dn = (((0,), (1,)), ((), ()))

    for s, (rc_ref, em_ref) in enumerate(((rc0_ref, em0_ref),
                                          (rc1_ref, em1_ref))):
        acc = jnp.zeros((_D, _TN), jnp.float32)
        for kb in range(_E // _TK):
            eo = kb * _TK
            rcv = rc_ref[0:1, eo:eo + _TK]
            onehot = jnp.where(row_ids == rcv, one, zero)
            acc = acc + jax.lax.dot_general(
                em_ref[eo:eo + _TK, :], onehot, dn,
                preferred_element_type=jnp.float32)
        xcat_ref[(s + 1) * _D:(s + 2) * _D, :] = acc.astype(jnp.bfloat16)

    xt = ndt_ref[...]
    xcat_ref[0:_D, :] = xt.astype(jnp.bfloat16)

    dt = (((0,), (0,)), ((), ()))
    h = jax.lax.dot_general(w1_ref[...], xcat_ref[...], dt,
                            preferred_element_type=jnp.float32) + b1t_ref[...]
    h = jnp.maximum(h, 0.0)
    y = jax.lax.dot_general(w2_ref[...], h.astype(jnp.bfloat16), dt,
                            preferred_element_type=jnp.float32) + b2t_ref[...]
    mu = jnp.mean(y, axis=0, keepdims=True)
    var = jnp.maximum(jnp.mean(y * y, axis=0, keepdims=True) - mu * mu, 0.0)
    yn = (y - mu) * jax.lax.rsqrt(var + 1e-5)
    ot_ref[...] = yn * gt_ref[...] + btt_ref[...] + xt


def _node_update(ndt, rc0, rc1, em0, em1, w1, b1t, w2, b2t, gt, btt):
    def full2(shp):
        return pl.BlockSpec(shp, lambda i: (0, 0))
    return pl.pallas_call(
        _node_kernel,
        out_shape=jax.ShapeDtypeStruct((_D, _N), jnp.float32),
        grid=(_N // _TN,),
        in_specs=[
            pl.BlockSpec((_D, _TN), lambda i: (0, i)),
            full2((1, _E)), full2((1, _E)),
            full2((_E, _D)), full2((_E, _D)),
            full2((3 * _D, _D)),
            full2((_D, _TN)),
            full2((_D, _D)),
            full2((_D, _TN)),
            full2((_D, _TN)),
            full2((_D, _TN)),
        ],
        out_specs=pl.BlockSpec((_D, _TN), lambda i: (0, i)),
        scratch_shapes=[pltpu.VMEM((3 * _D, _TN), jnp.bfloat16)],
        compiler_params=pltpu.CompilerParams(
            dimension_semantics=("parallel",)),
    )(ndt, rc0, rc1, em0, em1, w1, b1t, w2, b2t, gt, btt)


def kernel(node_features, ef0, snd0, rcv0, ef1, snd1, rcv1,
           edge_w1, edge_b1, edge_w2, edge_b2, edge_gamma, edge_beta,
           node_w1, node_b1, node_w2, node_b2, node_gamma, node_beta):
    d = _D
    node_bf = node_features.astype(jnp.bfloat16)
    s0 = jnp.take(node_bf, snd0.astype(jnp.int32), axis=0)
    d0 = jnp.take(node_bf, rcv0.astype(jnp.int32), axis=0)
    s1 = jnp.take(node_bf, snd1.astype(jnp.int32), axis=0)
    d1 = jnp.take(node_bf, rcv1.astype(jnp.int32), axis=0)

    zw1 = jnp.zeros((3 * d, d), jnp.bfloat16)
    w1bd = jnp.concatenate([
        jnp.concatenate([edge_w1[0], zw1], axis=1),
        jnp.concatenate([zw1, edge_w1[1]], axis=1)], axis=0)
    zw2 = jnp.zeros((d, d), jnp.bfloat16)
    w2bd = jnp.concatenate([
        jnp.concatenate([edge_w2[0], zw2], axis=1),
        jnp.concatenate([zw2, edge_w2[1]], axis=1)], axis=0)
    b1c = jnp.concatenate([edge_b1[0], edge_b1[1]], axis=1)
    b2c = jnp.concatenate([edge_b2[0], edge_b2[1]], axis=1)
    gc = jnp.concatenate([edge_gamma[0], edge_gamma[1]], axis=1)
    btc = jnp.concatenate([edge_beta[0], edge_beta[1]], axis=1)

    em0, em1, ne0, ne1 = _edge_mlps(s0, d0, ef0, s1, d1, ef1,
                                    w1bd, b1c, w2bd, b2c, gc, btc)

    ndt = node_features.T
    rc0 = rcv0.astype(jnp.int16).reshape(1, _E)
    rc1 = rcv1.astype(jnp.int16).reshape(1, _E)
    b1t = jnp.broadcast_to(node_b1.reshape(d, 1), (d, _TN))
    b2t = jnp.broadcast_to(node_b2.reshape(d, 1), (d, _TN))
    gt = jnp.broadcast_to(node_gamma.reshape(d, 1), (d, _TN))
    btt = jnp.broadcast_to(node_beta.reshape(d, 1), (d, _TN))

    out_t = _node_update(ndt, rc0, rc1, em0, em1,
                         node_w1, b1t, node_w2, b2t, gt, btt)
    return out_t.T, [ne0, ne1]


# speedup vs baseline: 1.6250x; 1.2856x over previous
import jax
import jax.numpy as jnp
from jax.experimental import pallas as pl
from jax.experimental.pallas import tpu as pltpu

_N = 16384
_E = 32768
_D = 128

_TE_EDGE = 4096
_TN = 512
_TK = 2048


def _edge_kernel(s0_ref, d0_ref, e0_ref, s1_ref, d1_ref, e1_ref,
                 w1_ref, b1_ref, w2_ref, b2_ref, g_ref, bt_ref,
                 p0_ref, p1_ref, r0_ref, r1_ref, xcat_ref):
    d = _D
    e0f = e0_ref[...]
    e1f = e1_ref[...]
    xcat_ref[:, 0:d] = s0_ref[...]
    xcat_ref[:, d:2 * d] = d0_ref[...]
    xcat_ref[:, 2 * d:3 * d] = e0f.astype(jnp.bfloat16)
    xcat_ref[:, 3 * d:4 * d] = s1_ref[...]
    xcat_ref[:, 4 * d:5 * d] = d1_ref[...]
    xcat_ref[:, 5 * d:6 * d] = e1f.astype(jnp.bfloat16)

    h = jnp.dot(xcat_ref[...], w1_ref[...],
                preferred_element_type=jnp.float32) + b1_ref[...]
    h = jnp.maximum(h, 0.0)
    y = jnp.dot(h.astype(jnp.bfloat16), w2_ref[...],
                preferred_element_type=jnp.float32) + b2_ref[...]

    for s, (p_ref, r_ref, ef) in enumerate(((p0_ref, r0_ref, e0f),
                                            (p1_ref, r1_ref, e1f))):
        ys = y[:, s * d:(s + 1) * d]
        mu = jnp.mean(ys, axis=-1, keepdims=True)
        var = jnp.maximum(jnp.mean(ys * ys, axis=-1, keepdims=True) - mu * mu,
                          0.0)
        ln = (ys - mu) * jax.lax.rsqrt(var + 1e-5) \
            * g_ref[:, s * d:(s + 1) * d] + bt_ref[:, s * d:(s + 1) * d]
        p_ref[...] = ln.astype(jnp.bfloat16).T
        r_ref[...] = ln + ef


def _edge_mlps(s0, d0, e0, s1, d1, e1, w1bd, b1c, w2bd, b2c, gc, btc):
    te = _TE_EDGE
    row = lambda ei: (ei, 0)
    return pl.pallas_call(
        _edge_kernel,
        out_shape=(jax.ShapeDtypeStruct((_D, _E), jnp.bfloat16),
                   jax.ShapeDtypeStruct((_D, _E), jnp.bfloat16),
                   jax.ShapeDtypeStruct((_E, _D), jnp.float32),
                   jax.ShapeDtypeStruct((_E, _D), jnp.float32)),
        grid=(_E // te,),
        in_specs=[pl.BlockSpec((te, _D), row)] * 6 + [
            pl.BlockSpec((6 * _D, 2 * _D), lambda ei: (0, 0)),
            pl.BlockSpec((1, 2 * _D), lambda ei: (0, 0)),
            pl.BlockSpec((2 * _D, 2 * _D), lambda ei: (0, 0)),
            pl.BlockSpec((1, 2 * _D), lambda ei: (0, 0)),
            pl.BlockSpec((1, 2 * _D), lambda ei: (0, 0)),
            pl.BlockSpec((1, 2 * _D), lambda ei: (0, 0)),
        ],
        out_specs=[pl.BlockSpec((_D, te), lambda ei: (0, ei))] * 2 +
                  [pl.BlockSpec((te, _D), row)] * 2,
        scratch_shapes=[pltpu.VMEM((te, 6 * _D), jnp.bfloat16)],
        compiler_params=pltpu.CompilerParams(
            dimension_semantics=("parallel",)),
    )(s0, d0, e0, s1, d1, e1, w1bd, b1c, w2bd, b2c, gc, btc)


def _node_kernel(ndt_ref, rc0_ref, rc1_ref, em0_ref, em1_ref,
                 w1_ref, b1t_ref, w2_ref, b2t_ref, gt_ref, btt_ref,
                 ot_ref, xcat_ref):
    n_base = pl.program_id(0) * _TN
    lane = jax.lax.broadcasted_iota(jnp.int32, (_TK, 128), 1)
    targets = [(lane + (n_base + g * 128)).astype(jnp.int16)
               for g in range(_TN // 128)]

    one = jnp.bfloat16(1.0)
    zero = jnp.bfloat16(0.0)

    for s, (rc_ref, em_ref) in enumerate(((rc0_ref, em0_ref),
                                          (rc1_ref, em1_ref))):
        acc = jnp.zeros((_D, _TN), jnp.float32)
        for kb in range(_E // _TK):
            eo = kb * _TK
            rr = rc_ref[eo:eo + _TK, :]
            onehot = jnp.concatenate(
                [jnp.where(rr == targets[g], one, zero)
                 for g in range(_TN // 128)], axis=1)
            acc = acc + jnp.dot(em_ref[:, eo:eo + _TK], onehot,
                                preferred_element_type=jnp.float32)
        xcat_ref[(s + 1) * _D:(s + 2) * _D, :] = acc.astype(jnp.bfloat16)

    xt = ndt_ref[...]
    xcat_ref[0:_D, :] = xt.astype(jnp.bfloat16)

    dt = (((0,), (0,)), ((), ()))
    h = jax.lax.dot_general(w1_ref[...], xcat_ref[...], dt,
                            preferred_element_type=jnp.float32) + b1t_ref[...]
    h = jnp.maximum(h, 0.0)
    y = jax.lax.dot_general(w2_ref[...], h.astype(jnp.bfloat16), dt,
                            preferred_element_type=jnp.float32) + b2t_ref[...]
    mu = jnp.mean(y, axis=0, keepdims=True)
    var = jnp.maximum(jnp.mean(y * y, axis=0, keepdims=True) - mu * mu, 0.0)
    yn = (y - mu) * jax.lax.rsqrt(var + 1e-5)
    ot_ref[...] = yn * gt_ref[...] + btt_ref[...] + xt


def _node_update(ndt, rc0, rc1, em0, em1, w1, b1t, w2, b2t, gt, btt):
    def full2(shp):
        return pl.BlockSpec(shp, lambda i: (0, 0))
    return pl.pallas_call(
        _node_kernel,
        out_shape=jax.ShapeDtypeStruct((_D, _N), jnp.float32),
        grid=(_N // _TN,),
        in_specs=[
            pl.BlockSpec((_D, _TN), lambda i: (0, i)),
            full2((_E, 128)), full2((_E, 128)),
            full2((_D, _E)), full2((_D, _E)),
            full2((3 * _D, _D)),
            full2((_D, _TN)),
            full2((_D, _D)),
            full2((_D, _TN)),
            full2((_D, _TN)),
            full2((_D, _TN)),
        ],
        out_specs=pl.BlockSpec((_D, _TN), lambda i: (0, i)),
        scratch_shapes=[pltpu.VMEM((3 * _D, _TN), jnp.bfloat16)],
        compiler_params=pltpu.CompilerParams(
            dimension_semantics=("parallel",)),
    )(ndt, rc0, rc1, em0, em1, w1, b1t, w2, b2t, gt, btt)


def kernel(node_features, ef0, snd0, rcv0, ef1, snd1, rcv1,
           edge_w1, edge_b1, edge_w2, edge_b2, edge_gamma, edge_beta,
           node_w1, node_b1, node_w2, node_b2, node_gamma, node_beta):
    d = _D
    node_bf = node_features.astype(jnp.bfloat16)
    s0 = jnp.take(node_bf, snd0.astype(jnp.int32), axis=0)
    d0 = jnp.take(node_bf, rcv0.astype(jnp.int32), axis=0)
    s1 = jnp.take(node_bf, snd1.astype(jnp.int32), axis=0)
    d1 = jnp.take(node_bf, rcv1.astype(jnp.int32), axis=0)

    zw1 = jnp.zeros((3 * d, d), jnp.bfloat16)
    w1bd = jnp.concatenate([
        jnp.concatenate([edge_w1[0], zw1], axis=1),
        jnp.concatenate([zw1, edge_w1[1]], axis=1)], axis=0)
    zw2 = jnp.zeros((d, d), jnp.bfloat16)
    w2bd = jnp.concatenate([
        jnp.concatenate([edge_w2[0], zw2], axis=1),
        jnp.concatenate([zw2, edge_w2[1]], axis=1)], axis=0)
    b1c = jnp.concatenate([edge_b1[0], edge_b1[1]], axis=1)
    b2c = jnp.concatenate([edge_b2[0], edge_b2[1]], axis=1)
    gc = jnp.concatenate([edge_gamma[0], edge_gamma[1]], axis=1)
    btc = jnp.concatenate([edge_beta[0], edge_beta[1]], axis=1)

    em0, em1, ne0, ne1 = _edge_mlps(s0, d0, ef0, s1, d1, ef1,
                                    w1bd, b1c, w2bd, b2c, gc, btc)

    ndt = node_features.T
    rc0 = jnp.broadcast_to(rcv0.astype(jnp.int16)[:, None], (_E, 128))
    rc1 = jnp.broadcast_to(rcv1.astype(jnp.int16)[:, None], (_E, 128))
    b1t = jnp.broadcast_to(node_b1.reshape(d, 1), (d, _TN))
    b2t = jnp.broadcast_to(node_b2.reshape(d, 1), (d, _TN))
    gt = jnp.broadcast_to(node_gamma.reshape(d, 1), (d, _TN))
    btt = jnp.broadcast_to(node_beta.reshape(d, 1), (d, _TN))

    out_t = _node_update(ndt, rc0, rc1, em0, em1,
                         node_w1, b1t, node_w2, b2t, gt, btt)
    return out_t.T, [ne0, ne1]


# speedup vs baseline: 2.2440x; 1.3809x over previous
import jax
import jax.numpy as jnp
from jax.experimental import pallas as pl
from jax.experimental.pallas import tpu as pltpu

_N = 16384
_E = 32768
_D = 128

_TE = 2048
_GU = 16
_TN = 512
_TK = 2048


def _edge_kernel(snd0_ref, rcv0_ref, snd1_ref, rcv1_ref,
                 ps0_ref, pd0_ref, ps1_ref, pd1_ref,
                 e0_ref, e1_ref,
                 we_ref, b1_ref, w2_ref, b2_ref, g_ref, bt_ref,
                 t0_ref, t1_ref, r0_ref, r1_ref,
                 ps0_v, pd0_v, ps1_v, pd1_v, g0_ref, g1_ref, sems):
    d = _D
    te = _TE
    n_in = _E // te // 2
    base = (pl.program_id(0) * n_in + pl.program_id(1)) * te

    @pl.when(pl.program_id(1) == 0)
    def _load_tables():
        for k, (src, dst) in enumerate(((ps0_ref, ps0_v), (pd0_ref, pd0_v),
                                        (ps1_ref, ps1_v), (pd1_ref, pd1_v))):
            pltpu.make_async_copy(src, dst, sems.at[k]).start()
        for k, (src, dst) in enumerate(((ps0_ref, ps0_v), (pd0_ref, pd0_v),
                                        (ps1_ref, ps1_v), (pd1_ref, pd1_v))):
            pltpu.make_async_copy(src, dst, sems.at[k]).wait()

    def gather_set(si_ref, di_ref, ps_ref, pd_ref, gb_ref):
        def chunk(c, carry):
            o = base + c * _GU
            rows = []
            for u in range(_GU):
                rows.append(ps_ref[si_ref[o + u], 0]
                            + pd_ref[di_ref[o + u], 0])
            for u in range(_GU):
                gb_ref[c * _GU + u, 0] = rows[u]
            return carry
        jax.lax.fori_loop(0, te // _GU, chunk, 0)

    gather_set(snd0_ref, rcv0_ref, ps0_v, pd0_v, g0_ref)
    gather_set(snd1_ref, rcv1_ref, ps1_v, pd1_v, g1_ref)

    for s, (gb_ref, e_ref, t_ref, r_ref) in enumerate(
            ((g0_ref, e0_ref, t0_ref, r0_ref),
             (g1_ref, e1_ref, t1_ref, r1_ref))):
        ef = e_ref[...]
        hp = jnp.dot(ef.astype(jnp.bfloat16), we_ref[s],
                     preferred_element_type=jnp.float32)
        h = gb_ref[...].reshape(te, d) + hp + b1_ref[s]
        h = jnp.maximum(h, 0.0)
        y = jnp.dot(h.astype(jnp.bfloat16), w2_ref[s],
                    preferred_element_type=jnp.float32) + b2_ref[s]
        mu = jnp.mean(y, axis=-1, keepdims=True)
        var = jnp.maximum(jnp.mean(y * y, axis=-1, keepdims=True) - mu * mu,
                          0.0)
        ln = (y - mu) * jax.lax.rsqrt(var + 1e-5) * g_ref[s] + bt_ref[s]
        t_ref[...] = ln.astype(jnp.bfloat16).T
        r_ref[...] = ln + ef


def _edge_mlps(snd0, rcv0, snd1, rcv1, ps0, pd0, ps1, pd1, e0, e1,
               w1e, b1, w2, b2, g, bt):
    te = _TE
    n_in = _E // te // 2
    smem = pl.BlockSpec(memory_space=pltpu.SMEM)
    hbm = pl.BlockSpec(memory_space=pl.ANY)

    def fullb(shp):
        nd = len(shp)
        return pl.BlockSpec(shp, lambda c, t: (0,) * nd)

    def rowb(shp0):
        return lambda c, t: (c * n_in + t, 0)
    return pl.pallas_call(
        _edge_kernel,
        out_shape=(jax.ShapeDtypeStruct((_D, _E), jnp.bfloat16),
                   jax.ShapeDtypeStruct((_D, _E), jnp.bfloat16),
                   jax.ShapeDtypeStruct((_E, _D), jnp.float32),
                   jax.ShapeDtypeStruct((_E, _D), jnp.float32)),
        grid=(2, n_in),
        in_specs=[smem] * 4 + [hbm] * 4 + [
            pl.BlockSpec((te, _D), rowb(te)),
            pl.BlockSpec((te, _D), rowb(te)),
            fullb((2, _D, _D)),
            fullb((2, 1, _D)),
            fullb((2, _D, _D)),
            fullb((2, 1, _D)),
            fullb((2, 1, _D)),
            fullb((2, 1, _D)),
        ],
        out_specs=[pl.BlockSpec((_D, te), lambda c, t: (0, c * n_in + t))] * 2
                 + [pl.BlockSpec((te, _D), rowb(te))] * 2,
        scratch_shapes=[pltpu.VMEM((_N, 1, _D), jnp.float32)] * 4 +
                       [pltpu.VMEM((te, 1, _D), jnp.float32)] * 2 +
                       [pltpu.SemaphoreType.DMA((4,))],
        compiler_params=pltpu.CompilerParams(
            dimension_semantics=("parallel", "arbitrary")),
    )(snd0, rcv0, snd1, rcv1, ps0, pd0, ps1, pd1, e0, e1,
      w1e, b1, w2, b2, g, bt)


def _node_kernel(ndn_ref, rc0_ref, rc1_ref, em0_ref, em1_ref,
                 w1_ref, b1t_ref, w2_ref, b2t_ref, gt_ref, btt_ref,
                 on_ref, rc0_v, rc1_v, em0_v, em1_v, xcat_ref, sems):
    n_in = _N // _TN // 2
    n_base = (pl.program_id(0) * n_in + pl.program_id(1)) * _TN

    @pl.when(pl.program_id(1) == 0)
    def _load_big():
        for k, (src, dst) in enumerate(((rc0_ref, rc0_v), (rc1_ref, rc1_v),
                                        (em0_ref, em0_v), (em1_ref, em1_v))):
            pltpu.make_async_copy(src, dst, sems.at[k]).start()
        for k, (src, dst) in enumerate(((rc0_ref, rc0_v), (rc1_ref, rc1_v),
                                        (em0_ref, em0_v), (em1_ref, em1_v))):
            pltpu.make_async_copy(src, dst, sems.at[k]).wait()

    lane = jax.lax.broadcasted_iota(jnp.int32, (_TK, 128), 1)
    targets = [(lane + (n_base + g * 128)).astype(jnp.int16)
               for g in range(_TN // 128)]

    one = jnp.bfloat16(1.0)
    zero = jnp.bfloat16(0.0)

    for s, (rc_v, em_v) in enumerate(((rc0_v, em0_v), (rc1_v, em1_v))):
        acc = jnp.zeros((_D, _TN), jnp.float32)
        for kb in range(_E // _TK):
            eo = kb * _TK
            rr = rc_v[eo:eo + _TK, :]
            onehot = jnp.concatenate(
                [jnp.where(rr == targets[g], one, zero)
                 for g in range(_TN // 128)], axis=1)
            acc = acc + jnp.dot(em_v[:, eo:eo + _TK], onehot,
                                preferred_element_type=jnp.float32)
        xcat_ref[(s + 1) * _D:(s + 2) * _D, :] = acc.astype(jnp.bfloat16)

    xt = ndn_ref[...].T
    xcat_ref[0:_D, :] = xt.astype(jnp.bfloat16)

    dt = (((0,), (0,)), ((), ()))
    h = jax.lax.dot_general(w1_ref[...], xcat_ref[...], dt,
                            preferred_element_type=jnp.float32) + b1t_ref[...]
    h = jnp.maximum(h, 0.0)
    y = jax.lax.dot_general(w2_ref[...], h.astype(jnp.bfloat16), dt,
                            preferred_element_type=jnp.float32) + b2t_ref[...]
    mu = jnp.mean(y, axis=0, keepdims=True)
    var = jnp.maximum(jnp.mean(y * y, axis=0, keepdims=True) - mu * mu, 0.0)
    yn = (y - mu) * jax.lax.rsqrt(var + 1e-5)
    on_ref[...] = (yn * gt_ref[...] + btt_ref[...] + xt).T


def _node_update(ndn, rc0, rc1, em0, em1, w1, b1t, w2, b2t, gt, btt):
    n_in = _N // _TN // 2
    hbm = pl.BlockSpec(memory_space=pl.ANY)

    def full2(shp):
        return pl.BlockSpec(shp, lambda c, t: (0, 0))
    return pl.pallas_call(
        _node_kernel,
        out_shape=jax.ShapeDtypeStruct((_N, _D), jnp.float32),
        grid=(2, n_in),
        in_specs=[
            pl.BlockSpec((_TN, _D), lambda c, t: (c * n_in + t, 0)),
            hbm, hbm,
            hbm, hbm,
            full2((3 * _D, _D)),
            full2((_D, _TN)),
            full2((_D, _D)),
            full2((_D, _TN)),
            full2((_D, _TN)),
            full2((_D, _TN)),
        ],
        out_specs=pl.BlockSpec((_TN, _D), lambda c, t: (c * n_in + t, 0)),
        scratch_shapes=[pltpu.VMEM((_E, 128), jnp.int16),
                        pltpu.VMEM((_E, 128), jnp.int16),
                        pltpu.VMEM((_D, _E), jnp.bfloat16),
                        pltpu.VMEM((_D, _E), jnp.bfloat16),
                        pltpu.VMEM((3 * _D, _TN), jnp.bfloat16),
                        pltpu.SemaphoreType.DMA((4,))],
        compiler_params=pltpu.CompilerParams(
            dimension_semantics=("parallel", "arbitrary")),
    )(ndn, rc0, rc1, em0, em1, w1, b1t, w2, b2t, gt, btt)


def kernel(node_features, ef0, snd0, rcv0, ef1, snd1, rcv1,
           edge_w1, edge_b1, edge_w2, edge_b2, edge_gamma, edge_beta,
           node_w1, node_b1, node_w2, node_b2, node_gamma, node_beta):
    d = _D
    node_bf = node_features.astype(jnp.bfloat16)

    def proj(wblock):
        return jnp.dot(node_bf, wblock,
                       preferred_element_type=jnp.float32).reshape(_N, 1, d)

    ps0 = proj(edge_w1[0, 0:d, :])
    pd0 = proj(edge_w1[0, d:2 * d, :])
    ps1 = proj(edge_w1[1, 0:d, :])
    pd1 = proj(edge_w1[1, d:2 * d, :])
    w1e = edge_w1[:, 2 * d:3 * d, :]

    em0, em1, ne0, ne1 = _edge_mlps(
        snd0.astype(jnp.int32), rcv0.astype(jnp.int32),
        snd1.astype(jnp.int32), rcv1.astype(jnp.int32),
        ps0, pd0, ps1, pd1, ef0, ef1,
        w1e, edge_b1, edge_w2, edge_b2, edge_gamma, edge_beta)

    rc0 = jnp.broadcast_to(rcv0.astype(jnp.int16)[:, None], (_E, 128))
    rc1 = jnp.broadcast_to(rcv1.astype(jnp.int16)[:, None], (_E, 128))
    b1t = jnp.broadcast_to(node_b1.reshape(d, 1), (d, _TN))
    b2t = jnp.broadcast_to(node_b2.reshape(d, 1), (d, _TN))
    gt = jnp.broadcast_to(node_gamma.reshape(d, 1), (d, _TN))
    btt = jnp.broadcast_to(node_beta.reshape(d, 1), (d, _TN))

    new_nodes = _node_update(node_features, rc0, rc1, em0, em1,
                             node_w1, b1t, node_w2, b2t, gt, btt)
    return new_nodes, [ne0, ne1]


# speedup vs baseline: 2.6877x; 1.1977x over previous
import jax
import jax.numpy as jnp
from jax.experimental import pallas as pl
from jax.experimental.pallas import tpu as pltpu

_N = 16384
_E = 32768
_D = 128

_TE = 2048
_GU = 16
_TN = 512
_TK = 2048


def _edge_kernel(snd0_ref, rcv0_ref, snd1_ref, rcv1_ref,
                 ps0_ref, pd0_ref, ps1_ref, pd1_ref,
                 e0_ref, e1_ref,
                 we_ref, b1_ref, w2_ref, b2_ref, g_ref, bt_ref,
                 t0_ref, t1_ref, r0_ref, r1_ref,
                 g0_ref, g1_ref):
    d = _D
    te = _TE
    base = pl.program_id(0) * te

    def gather_set(si_ref, di_ref, ps_ref, pd_ref, gb_ref):
        def chunk(c, carry):
            o = base + c * _GU
            rows = []
            for u in range(_GU):
                rows.append(ps_ref[si_ref[o + u], 0]
                            + pd_ref[di_ref[o + u], 0])
            for u in range(_GU):
                gb_ref[c * _GU + u, 0] = rows[u]
            return carry
        jax.lax.fori_loop(0, te // _GU, chunk, 0)

    gather_set(snd0_ref, rcv0_ref, ps0_ref, pd0_ref, g0_ref)
    gather_set(snd1_ref, rcv1_ref, ps1_ref, pd1_ref, g1_ref)

    for s, (gb_ref, e_ref, t_ref, r_ref) in enumerate(
            ((g0_ref, e0_ref, t0_ref, r0_ref),
             (g1_ref, e1_ref, t1_ref, r1_ref))):
        ef = e_ref[...]
        hp = jnp.dot(ef.astype(jnp.bfloat16), we_ref[s],
                     preferred_element_type=jnp.float32)
        h = gb_ref[...].reshape(te, d) + hp + b1_ref[s]
        h = jnp.maximum(h, 0.0)
        y = jnp.dot(h.astype(jnp.bfloat16), w2_ref[s],
                    preferred_element_type=jnp.float32) + b2_ref[s]
        mu = jnp.mean(y, axis=-1, keepdims=True)
        var = jnp.maximum(jnp.mean(y * y, axis=-1, keepdims=True) - mu * mu,
                          0.0)
        ln = (y - mu) * jax.lax.rsqrt(var + 1e-5) * g_ref[s] + bt_ref[s]
        t_ref[...] = ln.astype(jnp.bfloat16).T
        r_ref[...] = ln + ef


def _edge_mlps(snd0, rcv0, snd1, rcv1, ps0, pd0, ps1, pd1, e0, e1,
               w1e, b1, w2, b2, g, bt):
    te = _TE
    smem = pl.BlockSpec(memory_space=pltpu.SMEM)

    def fullb(shp):
        nd = len(shp)
        return pl.BlockSpec(shp, lambda ei: (0,) * nd)
    return pl.pallas_call(
        _edge_kernel,
        out_shape=(jax.ShapeDtypeStruct((_D, _E), jnp.bfloat16),
                   jax.ShapeDtypeStruct((_D, _E), jnp.bfloat16),
                   jax.ShapeDtypeStruct((_E, _D), jnp.float32),
                   jax.ShapeDtypeStruct((_E, _D), jnp.float32)),
        grid=(_E // te,),
        in_specs=[smem] * 4 + [fullb((_N, 1, _D))] * 4 + [
            pl.BlockSpec((te, _D), lambda ei: (ei, 0)),
            pl.BlockSpec((te, _D), lambda ei: (ei, 0)),
            fullb((2, _D, _D)),
            fullb((2, 1, _D)),
            fullb((2, _D, _D)),
            fullb((2, 1, _D)),
            fullb((2, 1, _D)),
            fullb((2, 1, _D)),
        ],
        out_specs=[pl.BlockSpec((_D, te), lambda ei: (0, ei))] * 2 +
                  [pl.BlockSpec((te, _D), lambda ei: (ei, 0))] * 2,
        scratch_shapes=[pltpu.VMEM((te, 1, _D), jnp.float32),
                        pltpu.VMEM((te, 1, _D), jnp.float32)],
        compiler_params=pltpu.CompilerParams(
            dimension_semantics=("parallel",)),
    )(snd0, rcv0, snd1, rcv1, ps0, pd0, ps1, pd1, e0, e1,
      w1e, b1, w2, b2, g, bt)


def _node_kernel(ndn_ref, rc0_ref, rc1_ref, em0_ref, em1_ref,
                 w1_ref, b1t_ref, w2_ref, b2t_ref, gt_ref, btt_ref,
                 on_ref, xcat_ref):
    n_base = pl.program_id(0) * _TN
    lane = jax.lax.broadcasted_iota(jnp.int32, (_TK, 128), 1)
    targets = [lane + (n_base + g * 128) for g in range(_TN // 128)]

    for s, (rc_ref, em_ref) in enumerate(((rc0_ref, em0_ref),
                                          (rc1_ref, em1_ref))):
        acc = jnp.zeros((_D, _TN), jnp.float32)
        for kb in range(_E // _TK):
            eo = kb * _TK
            rr = rc_ref[eo:eo + _TK, :]
            onehot = jnp.concatenate(
                [jnp.where(rr == targets[g], 1.0, 0.0)
                 for g in range(_TN // 128)], axis=1)
            acc = acc + jnp.dot(
                em_ref[:, eo:eo + _TK].astype(jnp.float32), onehot,
                preferred_element_type=jnp.float32)
        xcat_ref[(s + 1) * _D:(s + 2) * _D, :] = acc.astype(jnp.bfloat16)

    xt = ndn_ref[...].T
    xcat_ref[0:_D, :] = xt.astype(jnp.bfloat16)

    dt = (((0,), (0,)), ((), ()))
    h = jax.lax.dot_general(w1_ref[...], xcat_ref[...], dt,
                            preferred_element_type=jnp.float32) + b1t_ref[...]
    h = jnp.maximum(h, 0.0)
    y = jax.lax.dot_general(w2_ref[...], h.astype(jnp.bfloat16), dt,
                            preferred_element_type=jnp.float32) + b2t_ref[...]
    mu = jnp.mean(y, axis=0, keepdims=True)
    var = jnp.maximum(jnp.mean(y * y, axis=0, keepdims=True) - mu * mu, 0.0)
    yn = (y - mu) * jax.lax.rsqrt(var + 1e-5)
    on_ref[...] = (yn * gt_ref[...] + btt_ref[...] + xt).T


def _node_update(ndn, rc0, rc1, em0, em1, w1, b1t, w2, b2t, gt, btt):
    def full2(shp):
        return pl.BlockSpec(shp, lambda i: (0, 0))
    return pl.pallas_call(
        _node_kernel,
        out_shape=jax.ShapeDtypeStruct((_N, _D), jnp.float32),
        grid=(_N // _TN,),
        in_specs=[
            pl.BlockSpec((_TN, _D), lambda i: (i, 0)),
            full2((_E, 128)), full2((_E, 128)),
            full2((_D, _E)), full2((_D, _E)),
            full2((3 * _D, _D)),
            full2((_D, _TN)),
            full2((_D, _D)),
            full2((_D, _TN)),
            full2((_D, _TN)),
            full2((_D, _TN)),
        ],
        out_specs=pl.BlockSpec((_TN, _D), lambda i: (i, 0)),
        scratch_shapes=[pltpu.VMEM((3 * _D, _TN), jnp.bfloat16)],
        compiler_params=pltpu.CompilerParams(
            dimension_semantics=("parallel",)),
    )(ndn, rc0, rc1, em0, em1, w1, b1t, w2, b2t, gt, btt)


def kernel(node_features, ef0, snd0, rcv0, ef1, snd1, rcv1,
           edge_w1, edge_b1, edge_w2, edge_b2, edge_gamma, edge_beta,
           node_w1, node_b1, node_w2, node_b2, node_gamma, node_beta):
    d = _D
    node_bf = node_features.astype(jnp.bfloat16)

    def proj(wblock):
        return jnp.dot(node_bf, wblock,
                       preferred_element_type=jnp.float32).reshape(_N, 1, d)

    ps0 = proj(edge_w1[0, 0:d, :])
    pd0 = proj(edge_w1[0, d:2 * d, :])
    ps1 = proj(edge_w1[1, 0:d, :])
    pd1 = proj(edge_w1[1, d:2 * d, :])
    w1e = edge_w1[:, 2 * d:3 * d, :]

    em0, em1, ne0, ne1 = _edge_mlps(
        snd0.astype(jnp.int32), rcv0.astype(jnp.int32),
        snd1.astype(jnp.int32), rcv1.astype(jnp.int32),
        ps0, pd0, ps1, pd1, ef0, ef1,
        w1e, edge_b1, edge_w2, edge_b2, edge_gamma, edge_beta)

    rc0 = jnp.broadcast_to(rcv0.astype(jnp.int32)[:, None], (_E, 128))
    rc1 = jnp.broadcast_to(rcv1.astype(jnp.int32)[:, None], (_E, 128))
    b1t = jnp.broadcast_to(node_b1.reshape(d, 1), (d, _TN))
    b2t = jnp.broadcast_to(node_b2.reshape(d, 1), (d, _TN))
    gt = jnp.broadcast_to(node_gamma.reshape(d, 1), (d, _TN))
    btt = jnp.broadcast_to(node_beta.reshape(d, 1), (d, _TN))

    new_nodes = _node_update(node_features, rc0, rc1, em0, em1,
                             node_w1, b1t, node_w2, b2t, gt, btt)
    return new_nodes, [ne0, ne1]
